# baseline (speedup 1.0000x reference)
import functools

import jax
import jax.numpy as jnp
from jax import lax
from jax.experimental import pallas as pl
from jax.experimental.pallas import tpu as pltpu
from jax.experimental.pallas import tpu_sc as plsc

NC = 2
NS = 16
G = 64
DW = 16
CH = 400

_MESH = plsc.VectorSubcoreMesh(
    core_axis_name="core", subcore_axis_name="subcore",
    num_cores=NC, num_subcores=NS)


def _edge_window(e):
    for w in (256, 128, 64, 32, 16, 8):
        if e % w == 0:
            return w
    return 8


def _sc_degree(dst, ones_rows, zero_rows, n):
    e = dst.shape[0]
    d = ones_rows.shape[1]
    w = ones_rows.shape[0]
    nwin = e // w
    wpass = -(-nwin // (NC * NS))
    nch = n // CH
    npass = -(-nch // NS)

    @functools.partial(
        pl.kernel,
        out_type=jax.ShapeDtypeStruct((NC, n, d), jnp.float32),
        mesh=_MESH,
        scratch_types=[
            pltpu.VMEM((w,), jnp.int32),
            pltpu.VMEM((w, d), jnp.float32),
            pltpu.VMEM_SHARED((n, d), jnp.float32),
            pltpu.SemaphoreType.DMA,
        ])
    def deg_kernel(dst_hbm, ones_hbm, zero_hbm, out_hbm,
                   dst_v, ones_v, acc, sem):
        core = lax.axis_index("core")
        sub = lax.axis_index("subcore")
        wid = core * NS + sub
        pltpu.sync_copy(ones_hbm, ones_v)

        @pl.loop(0, npass)
        def _(c):
            chunk = c * NS + sub

            @pl.when(chunk < nch)
            def _():
                pltpu.sync_copy(zero_hbm, acc.at[pl.ds(chunk * CH, CH)])

        plsc.subcore_barrier()

        @pl.loop(0, wpass)
        def _(t):
            win = t * (NC * NS) + wid

            @pl.when(win < nwin)
            def _():
                pltpu.sync_copy(dst_hbm.at[pl.ds(win * w, w)], dst_v)
                pltpu.sync_copy(ones_v, acc.at[dst_v], add=True)

        plsc.subcore_barrier()

        @pl.loop(0, npass)
        def _(c):
            chunk = c * NS + sub

            @pl.when(chunk < nch)
            def _():
                pltpu.sync_copy(acc.at[pl.ds(chunk * CH, CH)],
                                out_hbm.at[core].at[pl.ds(chunk * CH, CH)])

    return deg_kernel(dst, ones_rows, zero_rows)


def _sc_aggregate(g, src_p, dst_p, zero_rows):
    n, d = g.shape
    e = dst_p.shape[0]
    w = 160
    assert e % w == 0
    nwin = e // w
    nw_ = NC * NS
    wpass = -(-nwin // nw_)
    nch = n // CH
    npass = -(-nch // NS)

    @functools.partial(
        pl.kernel,
        out_type=jax.ShapeDtypeStruct((NC, n, d), jnp.float32),
        mesh=_MESH,
        scratch_types=[
            pltpu.VMEM((w,), jnp.int32),
            pltpu.VMEM((w,), jnp.int32),
            pltpu.VMEM((w,), jnp.int32),
            pltpu.VMEM((w,), jnp.int32),
            pltpu.VMEM((w, d), jnp.float32),
            pltpu.VMEM((w, d), jnp.float32),
            pltpu.VMEM_SHARED((n, d), jnp.float32),
            pltpu.SemaphoreType.DMA,
            pltpu.SemaphoreType.DMA,
            pltpu.SemaphoreType.DMA,
            pltpu.SemaphoreType.DMA,
            pltpu.SemaphoreType.DMA,
            pltpu.SemaphoreType.DMA,
            pltpu.SemaphoreType.DMA,
            pltpu.SemaphoreType.DMA,
        ])
    def agg_kernel(g_hbm, src_hbm, dst_hbm, zero_hbm, out_hbm,
                   src0, dst0, src1, dst1, rows0, rows1, acc,
                   ss0, sd0, sg0, sa0, ss1, sd1, sg1, sa1):
        core = lax.axis_index("core")
        sub = lax.axis_index("subcore")
        wid = core * NS + sub

        @pl.loop(0, npass)
        def _(c):
            chunk = c * NS + sub

            @pl.when(chunk < nch)
            def _():
                pltpu.sync_copy(zero_hbm, acc.at[pl.ds(chunk * CH, CH)])

        plsc.subcore_barrier()

        def exists(k):
            return k * nw_ + wid < nwin

        def off(k):
            return (k * nw_ + wid) * w

        def issue_idx(k, src_b, dst_b, ssem, dsem):
            @pl.when(exists(k))
            def _():
                pltpu.async_copy(src_hbm.at[pl.ds(off(k), w)], src_b, ssem)
                pltpu.async_copy(dst_hbm.at[pl.ds(off(k), w)], dst_b, dsem)

        def issue_gather(k, src_b, rows_b, ssem, gsem):
            @pl.when(exists(k))
            def _():
                pltpu.make_async_copy(src_hbm.at[pl.ds(off(k), w)], src_b,
                                      ssem).wait()
                pltpu.async_copy(g_hbm.at[src_b], rows_b, gsem)

        def start_scatter(k, src_b, dst_b, rows_b, dsem, gsem, asem):
            @pl.when(exists(k))
            def _():
                pltpu.make_async_copy(g_hbm.at[src_b], rows_b, gsem).wait()
                pltpu.make_async_copy(dst_hbm.at[pl.ds(off(k), w)], dst_b,
                                      dsem).wait()
                pltpu.async_copy(rows_b, acc.at[dst_b], asem, add=True)

        def wait_scatter(k, dst_b, rows_b, asem):
            @pl.when(exists(k))
            def _():
                pltpu.make_async_copy(rows_b, acc.at[dst_b], asem).wait()

        issue_idx(0, src0, dst0, ss0, sd0)
        issue_idx(1, src1, dst1, ss1, sd1)
        issue_gather(0, src0, rows0, ss0, sg0)

        @pl.loop(0, (wpass + 1) // 2)
        def _(t):
            k = 2 * t
            issue_gather(k + 1, src1, rows1, ss1, sg1)
            start_scatter(k, src0, dst0, rows0, sd0, sg0, sa0)
            start_scatter(k + 1, src1, dst1, rows1, sd1, sg1, sa1)
            wait_scatter(k, dst0, rows0, sa0)
            issue_idx(k + 2, src0, dst0, ss0, sd0)
            issue_gather(k + 2, src0, rows0, ss0, sg0)
            wait_scatter(k + 1, dst1, rows1, sa1)
            issue_idx(k + 3, src1, dst1, ss1, sd1)

        plsc.subcore_barrier()

        @pl.loop(0, npass)
        def _(c):
            chunk = c * NS + sub

            @pl.when(chunk < nch)
            def _():
                pltpu.sync_copy(acc.at[pl.ds(chunk * CH, CH)],
                                out_hbm.at[core].at[pl.ds(chunk * CH, CH)])

    return agg_kernel(g, src_p, dst_p, zero_rows)


def _tc_first(x, w1, dg, blk):
    n, d = x.shape
    h = w1.shape[1]
    nb = n // blk

    def body(x_ref, w_ref, d0_ref, d1_ref, g_ref, dinv_ref):
        deg = d0_ref[0, :, 0:1] + d1_ref[0, :, 0:1] + 1.0
        dinv = lax.rsqrt(jnp.maximum(deg, 1.0))
        hh = jnp.dot(x_ref[...], w_ref[...],
                     preferred_element_type=jnp.float32, precision=lax.Precision.HIGHEST)
        g_ref[...] = hh * dinv
        dinv_ref[...] = jnp.broadcast_to(dinv, (blk, h))

    return pl.pallas_call(
        body,
        grid=(nb,),
        in_specs=[
            pl.BlockSpec((blk, d), lambda i: (i, 0)),
            pl.BlockSpec((d, h), lambda i: (0, 0)),
            pl.BlockSpec((1, blk, h), lambda i: (0, i, 0)),
            pl.BlockSpec((1, blk, h), lambda i: (1, i, 0)),
        ],
        out_specs=[
            pl.BlockSpec((blk, h), lambda i: (i, 0)),
            pl.BlockSpec((blk, h), lambda i: (i, 0)),
        ],
        out_shape=[
            jax.ShapeDtypeStruct((n, h), jnp.float32),
            jax.ShapeDtypeStruct((n, h), jnp.float32),
        ])(x, w1, dg, dg)


def _tc_layer(acc, g_prev, dinv, b, wk, blk):
    n, h = g_prev.shape

    def body(a0_ref, a1_ref, gp_ref, dinv_ref, b_ref, w_ref, g_ref):
        s = (a0_ref[0] + a1_ref[0] + gp_ref[...]) * dinv_ref[...] + b_ref[...]
        xk = jnp.maximum(s, 0.0)
        hh = jnp.dot(xk, w_ref[...], preferred_element_type=jnp.float32, precision=lax.Precision.HIGHEST)
        g_ref[...] = hh * dinv_ref[...]

    return pl.pallas_call(
        body,
        grid=(n // blk,),
        in_specs=[
            pl.BlockSpec((1, blk, h), lambda i: (0, i, 0)),
            pl.BlockSpec((1, blk, h), lambda i: (1, i, 0)),
            pl.BlockSpec((blk, h), lambda i: (i, 0)),
            pl.BlockSpec((blk, h), lambda i: (i, 0)),
            pl.BlockSpec((1, h), lambda i: (0, 0)),
            pl.BlockSpec((h, h), lambda i: (0, 0)),
        ],
        out_specs=pl.BlockSpec((blk, h), lambda i: (i, 0)),
        out_shape=jax.ShapeDtypeStruct((n, h), jnp.float32),
    )(acc, acc, g_prev, dinv, b, wk)


def _tc_final(acc, g4, dinv, b4, batch3d, wl, bl, blk):
    n, h = g4.shape
    nb = n // blk

    def body(a0_ref, a1_ref, g_ref, dinv_ref, b_ref, batch_ref,
             wl_ref, bl_ref, out_ref, pooled_ref):
        i = pl.program_id(0)
        h4 = (a0_ref[0] + a1_ref[0] + g_ref[...]) * dinv_ref[...] + b_ref[...]
        ids = jnp.broadcast_to(batch_ref[0], (G, blk))
        onehot = (lax.broadcasted_iota(jnp.int32, (G, blk), 0)
                  == ids).astype(jnp.float32)
        part = jnp.dot(onehot, h4, preferred_element_type=jnp.float32, precision=lax.Precision.HIGHEST)

        @pl.when(i == 0)
        def _():
            pooled_ref[...] = part

        @pl.when(i > 0)
        def _():
            pooled_ref[...] += part

        @pl.when(i == nb - 1)
        def _():
            out_ref[...] = (jnp.dot(pooled_ref[...], wl_ref[...],
                                    preferred_element_type=jnp.float32, precision=lax.Precision.HIGHEST)
                            + bl_ref[...])

    return pl.pallas_call(
        body,
        grid=(nb,),
        in_specs=[
            pl.BlockSpec((1, blk, h), lambda i: (0, i, 0)),
            pl.BlockSpec((1, blk, h), lambda i: (1, i, 0)),
            pl.BlockSpec((blk, h), lambda i: (i, 0)),
            pl.BlockSpec((blk, h), lambda i: (i, 0)),
            pl.BlockSpec((1, h), lambda i: (0, 0)),
            pl.BlockSpec((1, 1, blk), lambda i: (i, 0, 0)),
            pl.BlockSpec((h, 1), lambda i: (0, 0)),
            pl.BlockSpec((1, 1), lambda i: (0, 0)),
        ],
        out_specs=pl.BlockSpec((G, 1), lambda i: (0, 0)),
        out_shape=jax.ShapeDtypeStruct((G, 1), jnp.float32),
        scratch_shapes=[pltpu.VMEM((G, h), jnp.float32)],
    )(acc, acc, g4, dinv, b4, batch3d, wl, bl)


def kernel(x, edge_index, batch, W1, b1, W2, b2, W3, b3, W4, b4, Wl, bl):
    n, d = x.shape
    h = W1.shape[1]
    e = edge_index.shape[1]
    src = edge_index[0]
    dst = edge_index[1]

    blk = 1000 if n % 1000 == 0 else 8
    w = _edge_window(e)


    ones_rows = jnp.ones((w, d), jnp.float32)
    zero_rows = jnp.zeros((CH, d), jnp.float32)
    batch3d = batch.reshape(n // blk, 1, blk)
    b1r = b1.reshape(1, h)
    b2r = b2.reshape(1, h)
    b3r = b3.reshape(1, h)
    b4r = b4.reshape(1, h)
    blr = bl.reshape(1, 1)

    dg = _sc_degree(dst, ones_rows, zero_rows, n)
    g1, dinv = _tc_first(x, W1, dg, blk)
    a1_ = _sc_aggregate(g1, src, dst, zero_rows)
    g2 = _tc_layer(a1_, g1, dinv, b1r, W2, blk)
    a2_ = _sc_aggregate(g2, src, dst, zero_rows)
    g3 = _tc_layer(a2_, g2, dinv, b2r, W3, blk)
    a3_ = _sc_aggregate(g3, src, dst, zero_rows)
    g4 = _tc_layer(a3_, g3, dinv, b3r, W4, blk)
    a4_ = _sc_aggregate(g4, src, dst, zero_rows)
    return _tc_final(a4_, g4, dinv, b4r, batch3d, Wl, blr, blk)


# speedup vs baseline: 1.1606x; 1.1606x over previous
import functools

import jax
import jax.numpy as jnp
from jax import lax
from jax.experimental import pallas as pl
from jax.experimental.pallas import tpu as pltpu
from jax.experimental.pallas import tpu_sc as plsc

NC = 2
NS = 16
G = 64
DW = 16
CH = 400

_MESH = plsc.VectorSubcoreMesh(
    core_axis_name="core", subcore_axis_name="subcore",
    num_cores=NC, num_subcores=NS)


def _edge_window(e):
    for w in (256, 128, 64, 32, 16, 8):
        if e % w == 0:
            return w
    return 8


def _sc_degree(dst, ones_rows, zero_rows, n):
    e = dst.shape[0]
    d = ones_rows.shape[1]
    w = ones_rows.shape[0]
    nwin = e // w
    wpass = -(-nwin // (NC * NS))
    nch = n // CH
    npass = -(-nch // NS)

    @functools.partial(
        pl.kernel,
        out_type=jax.ShapeDtypeStruct((NC, n, d), jnp.float32),
        mesh=_MESH,
        scratch_types=[
            pltpu.VMEM((w,), jnp.int32),
            pltpu.VMEM((w, d), jnp.float32),
            pltpu.VMEM_SHARED((n, d), jnp.float32),
            pltpu.SemaphoreType.DMA,
        ])
    def deg_kernel(dst_hbm, ones_hbm, zero_hbm, out_hbm,
                   dst_v, ones_v, acc, sem):
        core = lax.axis_index("core")
        sub = lax.axis_index("subcore")
        wid = core * NS + sub
        pltpu.sync_copy(ones_hbm, ones_v)

        @pl.loop(0, npass)
        def _(c):
            chunk = c * NS + sub

            @pl.when(chunk < nch)
            def _():
                pltpu.sync_copy(zero_hbm, acc.at[pl.ds(chunk * CH, CH)])

        plsc.subcore_barrier()

        @pl.loop(0, wpass)
        def _(t):
            win = t * (NC * NS) + wid

            @pl.when(win < nwin)
            def _():
                pltpu.sync_copy(dst_hbm.at[pl.ds(win * w, w)], dst_v)
                pltpu.sync_copy(ones_v, acc.at[dst_v], add=True)

        plsc.subcore_barrier()

        @pl.loop(0, npass)
        def _(c):
            chunk = c * NS + sub

            @pl.when(chunk < nch)
            def _():
                pltpu.sync_copy(acc.at[pl.ds(chunk * CH, CH)],
                                out_hbm.at[core].at[pl.ds(chunk * CH, CH)])

    return deg_kernel(dst, ones_rows, zero_rows)


def _sc_aggregate(g, src_p, dst_p, zero_rows):
    n, d = g.shape
    e = dst_p.shape[0]
    w = 160
    assert e % w == 0
    nwin = e // w
    nw_ = NC * NS
    wpass = -(-nwin // nw_)
    nch = n // CH
    npass = -(-nch // NS)

    @functools.partial(
        pl.kernel,
        out_type=jax.ShapeDtypeStruct((NC, n, d), jnp.float32),
        mesh=_MESH,
        scratch_types=[
            pltpu.VMEM((w,), jnp.int32),
            pltpu.VMEM((w,), jnp.int32),
            pltpu.VMEM((w,), jnp.int32),
            pltpu.VMEM((w,), jnp.int32),
            pltpu.VMEM((w, d), jnp.float32),
            pltpu.VMEM((w, d), jnp.float32),
            pltpu.VMEM_SHARED((n, d), jnp.float32),
            pltpu.SemaphoreType.DMA,
            pltpu.SemaphoreType.DMA,
            pltpu.SemaphoreType.DMA,
            pltpu.SemaphoreType.DMA,
            pltpu.SemaphoreType.DMA,
            pltpu.SemaphoreType.DMA,
            pltpu.SemaphoreType.DMA,
            pltpu.SemaphoreType.DMA,
        ])
    def agg_kernel(g_hbm, src_hbm, dst_hbm, zero_hbm, out_hbm,
                   src0, dst0, src1, dst1, rows0, rows1, acc,
                   ss0, sd0, sg0, sa0, ss1, sd1, sg1, sa1):
        core = lax.axis_index("core")
        sub = lax.axis_index("subcore")
        wid = core * NS + sub

        @pl.loop(0, npass)
        def _(c):
            chunk = c * NS + sub

            @pl.when(chunk < nch)
            def _():
                pltpu.sync_copy(zero_hbm, acc.at[pl.ds(chunk * CH, CH)])

        plsc.subcore_barrier()

        def exists(k):
            return k * nw_ + wid < nwin

        def off(k):
            return (k * nw_ + wid) * w

        def issue_idx(k, src_b, dst_b, ssem, dsem):
            @pl.when(exists(k))
            def _():
                pltpu.async_copy(src_hbm.at[pl.ds(off(k), w)], src_b, ssem)
                pltpu.async_copy(dst_hbm.at[pl.ds(off(k), w)], dst_b, dsem)

        def issue_gather(k, src_b, rows_b, ssem, gsem):
            @pl.when(exists(k))
            def _():
                pltpu.make_async_copy(src_hbm.at[pl.ds(off(k), w)], src_b,
                                      ssem).wait()
                pltpu.async_copy(g_hbm.at[src_b], rows_b, gsem)

        def start_scatter(k, src_b, dst_b, rows_b, dsem, gsem, asem):
            @pl.when(exists(k))
            def _():
                pltpu.make_async_copy(g_hbm.at[src_b], rows_b, gsem).wait()
                pltpu.make_async_copy(dst_hbm.at[pl.ds(off(k), w)], dst_b,
                                      dsem).wait()
                pltpu.async_copy(rows_b, acc.at[dst_b], asem, add=True)

        def wait_scatter(k, dst_b, rows_b, asem):
            @pl.when(exists(k))
            def _():
                pltpu.make_async_copy(rows_b, acc.at[dst_b], asem).wait()

        issue_idx(0, src0, dst0, ss0, sd0)
        issue_idx(1, src1, dst1, ss1, sd1)
        issue_gather(0, src0, rows0, ss0, sg0)

        @pl.loop(0, (wpass + 1) // 2)
        def _(t):
            k = 2 * t
            issue_gather(k + 1, src1, rows1, ss1, sg1)
            start_scatter(k, src0, dst0, rows0, sd0, sg0, sa0)
            wait_scatter(k, dst0, rows0, sa0)
            issue_idx(k + 2, src0, dst0, ss0, sd0)
            issue_gather(k + 2, src0, rows0, ss0, sg0)
            start_scatter(k + 1, src1, dst1, rows1, sd1, sg1, sa1)
            wait_scatter(k + 1, dst1, rows1, sa1)
            issue_idx(k + 3, src1, dst1, ss1, sd1)

        plsc.subcore_barrier()

        @pl.loop(0, npass)
        def _(c):
            chunk = c * NS + sub

            @pl.when(chunk < nch)
            def _():
                pltpu.sync_copy(acc.at[pl.ds(chunk * CH, CH)],
                                out_hbm.at[core].at[pl.ds(chunk * CH, CH)])

    return agg_kernel(g, src_p, dst_p, zero_rows)


def _tc_first(x, w1, dg, blk):
    n, d = x.shape
    h = w1.shape[1]
    nb = n // blk

    def body(x_ref, w_ref, d0_ref, d1_ref, g_ref, dinv_ref):
        deg = d0_ref[0, :, 0:1] + d1_ref[0, :, 0:1] + 1.0
        dinv = lax.rsqrt(jnp.maximum(deg, 1.0))
        hh = jnp.dot(x_ref[...], w_ref[...],
                     preferred_element_type=jnp.float32, precision=lax.Precision.HIGHEST)
        g_ref[...] = hh * dinv
        dinv_ref[...] = jnp.broadcast_to(dinv, (blk, h))

    return pl.pallas_call(
        body,
        grid=(nb,),
        in_specs=[
            pl.BlockSpec((blk, d), lambda i: (i, 0)),
            pl.BlockSpec((d, h), lambda i: (0, 0)),
            pl.BlockSpec((1, blk, h), lambda i: (0, i, 0)),
            pl.BlockSpec((1, blk, h), lambda i: (1, i, 0)),
        ],
        out_specs=[
            pl.BlockSpec((blk, h), lambda i: (i, 0)),
            pl.BlockSpec((blk, h), lambda i: (i, 0)),
        ],
        out_shape=[
            jax.ShapeDtypeStruct((n, h), jnp.float32),
            jax.ShapeDtypeStruct((n, h), jnp.float32),
        ])(x, w1, dg, dg)


def _tc_layer(acc, g_prev, dinv, b, wk, blk):
    n, h = g_prev.shape

    def body(a0_ref, a1_ref, gp_ref, dinv_ref, b_ref, w_ref, g_ref):
        s = (a0_ref[0] + a1_ref[0] + gp_ref[...]) * dinv_ref[...] + b_ref[...]
        xk = jnp.maximum(s, 0.0)
        hh = jnp.dot(xk, w_ref[...], preferred_element_type=jnp.float32, precision=lax.Precision.HIGHEST)
        g_ref[...] = hh * dinv_ref[...]

    return pl.pallas_call(
        body,
        grid=(n // blk,),
        in_specs=[
            pl.BlockSpec((1, blk, h), lambda i: (0, i, 0)),
            pl.BlockSpec((1, blk, h), lambda i: (1, i, 0)),
            pl.BlockSpec((blk, h), lambda i: (i, 0)),
            pl.BlockSpec((blk, h), lambda i: (i, 0)),
            pl.BlockSpec((1, h), lambda i: (0, 0)),
            pl.BlockSpec((h, h), lambda i: (0, 0)),
        ],
        out_specs=pl.BlockSpec((blk, h), lambda i: (i, 0)),
        out_shape=jax.ShapeDtypeStruct((n, h), jnp.float32),
    )(acc, acc, g_prev, dinv, b, wk)


def _tc_final(acc, g4, dinv, b4, batch3d, wl, bl, blk):
    n, h = g4.shape
    nb = n // blk

    def body(a0_ref, a1_ref, g_ref, dinv_ref, b_ref, batch_ref,
             wl_ref, bl_ref, out_ref, pooled_ref):
        i = pl.program_id(0)
        h4 = (a0_ref[0] + a1_ref[0] + g_ref[...]) * dinv_ref[...] + b_ref[...]
        ids = jnp.broadcast_to(batch_ref[0], (G, blk))
        onehot = (lax.broadcasted_iota(jnp.int32, (G, blk), 0)
                  == ids).astype(jnp.float32)
        part = jnp.dot(onehot, h4, preferred_element_type=jnp.float32, precision=lax.Precision.HIGHEST)

        @pl.when(i == 0)
        def _():
            pooled_ref[...] = part

        @pl.when(i > 0)
        def _():
            pooled_ref[...] += part

        @pl.when(i == nb - 1)
        def _():
            out_ref[...] = (jnp.dot(pooled_ref[...], wl_ref[...],
                                    preferred_element_type=jnp.float32, precision=lax.Precision.HIGHEST)
                            + bl_ref[...])

    return pl.pallas_call(
        body,
        grid=(nb,),
        in_specs=[
            pl.BlockSpec((1, blk, h), lambda i: (0, i, 0)),
            pl.BlockSpec((1, blk, h), lambda i: (1, i, 0)),
            pl.BlockSpec((blk, h), lambda i: (i, 0)),
            pl.BlockSpec((blk, h), lambda i: (i, 0)),
            pl.BlockSpec((1, h), lambda i: (0, 0)),
            pl.BlockSpec((1, 1, blk), lambda i: (i, 0, 0)),
            pl.BlockSpec((h, 1), lambda i: (0, 0)),
            pl.BlockSpec((1, 1), lambda i: (0, 0)),
        ],
        out_specs=pl.BlockSpec((G, 1), lambda i: (0, 0)),
        out_shape=jax.ShapeDtypeStruct((G, 1), jnp.float32),
        scratch_shapes=[pltpu.VMEM((G, h), jnp.float32)],
    )(acc, acc, g4, dinv, b4, batch3d, wl, bl)


def kernel(x, edge_index, batch, W1, b1, W2, b2, W3, b3, W4, b4, Wl, bl):
    n, d = x.shape
    h = W1.shape[1]
    e = edge_index.shape[1]
    src = edge_index[0]
    dst = edge_index[1]

    blk = 1000 if n % 1000 == 0 else 8
    w = _edge_window(e)


    ones_rows = jnp.ones((w, d), jnp.float32)
    zero_rows = jnp.zeros((CH, d), jnp.float32)
    batch3d = batch.reshape(n // blk, 1, blk)
    b1r = b1.reshape(1, h)
    b2r = b2.reshape(1, h)
    b3r = b3.reshape(1, h)
    b4r = b4.reshape(1, h)
    blr = bl.reshape(1, 1)

    dg = _sc_degree(dst, ones_rows, zero_rows, n)
    g1, dinv = _tc_first(x, W1, dg, blk)
    a1_ = _sc_aggregate(g1, src, dst, zero_rows)
    g2 = _tc_layer(a1_, g1, dinv, b1r, W2, blk)
    a2_ = _sc_aggregate(g2, src, dst, zero_rows)
    g3 = _tc_layer(a2_, g2, dinv, b2r, W3, blk)
    a3_ = _sc_aggregate(g3, src, dst, zero_rows)
    g4 = _tc_layer(a3_, g3, dinv, b3r, W4, blk)
    a4_ = _sc_aggregate(g4, src, dst, zero_rows)
    return _tc_final(a4_, g4, dinv, b4r, batch3d, Wl, blr, blk)


# speedup vs baseline: 1.2222x; 1.0531x over previous
import dataclasses
import functools

import jax
import jax.numpy as jnp
from jax import lax
from jax.experimental import pallas as pl
from jax.experimental.pallas import tpu as pltpu
from jax.experimental.pallas import tpu_sc as plsc

NC = 2
NS = 16
G = 64
DW = 16
CH = 400

_MESH = plsc.VectorSubcoreMesh(
    core_axis_name="core", subcore_axis_name="subcore",
    num_cores=NC, num_subcores=NS)


def _edge_window(e):
    for w in (256, 128, 64, 32, 16, 8):
        if e % w == 0:
            return w
    return 8


def _sc_degree(dst, n_pad):
    e = dst.shape[0]
    epw = e // (NC * NS)
    rng = n_pad // NS
    nred = rng // 16

    cp = pltpu.CompilerParams()
    if "needs_layout_passes" in pltpu.CompilerParams.__dataclass_fields__:
        cp = dataclasses.replace(cp, needs_layout_passes=False)

    @functools.partial(
        pl.kernel,
        out_type=jax.ShapeDtypeStruct((NC, n_pad), jnp.float32),
        mesh=_MESH,
        compiler_params=cp,
        scratch_types=[
            pltpu.VMEM((epw,), jnp.int32),
            pltpu.VMEM((8, n_pad), jnp.float32),
            pltpu.VMEM((rng,), jnp.float32),
            pltpu.VMEM((rng,), jnp.float32),
            pltpu.VMEM_SHARED((NS, n_pad), jnp.float32),
            pltpu.SemaphoreType.DMA,
        ])
    def deg_kernel(dst_hbm, out_hbm,
                   didx, hist8, acc_r, tmp_r, stage, sem):
        core = lax.axis_index("core")
        sub = lax.axis_index("subcore")
        wid = core * NS + sub
        ones16 = jnp.ones((16,), jnp.float32)
        zeros16 = jnp.zeros((16,), jnp.float32)
        lane = lax.iota(jnp.int32, 16)
        row = lane % 8
        lo = lane < 8
        hi = jnp.logical_not(lo)

        @pl.loop(0, 8)
        def _(b):
            @pl.loop(0, n_pad // 16)
            def _(c):
                hist8[b, pl.ds(c * 16, 16)] = zeros16

        pltpu.sync_copy(dst_hbm.at[pl.ds(wid * epw, epw)], didx)

        @pl.loop(0, epw // 16)
        def _(i):
            idxv = didx[pl.ds(i * 16, 16)]
            plsc.addupdate_scatter(hist8, [row, idxv], ones16, mask=lo)
            plsc.addupdate_scatter(hist8, [row, idxv], ones16, mask=hi)

        @pl.loop(1, 8)
        def _(b):
            @pl.loop(0, n_pad // 16)
            def _(c):
                s = pl.ds(c * 16, 16)
                hist8[0, s] = hist8[0, s] + hist8[b, s]

        pltpu.sync_copy(hist8.at[0], stage.at[sub])
        plsc.subcore_barrier()

        pltpu.sync_copy(stage.at[0].at[pl.ds(sub * rng, rng)], acc_r)

        @pl.loop(1, NS)
        def _(j):
            pltpu.sync_copy(stage.at[j].at[pl.ds(sub * rng, rng)], tmp_r)

            @pl.loop(0, nred)
            def _(c):
                s = pl.ds(c * 16, 16)
                acc_r[s] = acc_r[s] + tmp_r[s]

        pltpu.sync_copy(acc_r, out_hbm.at[core].at[pl.ds(sub * rng, rng)])

    return deg_kernel(dst)


def _sc_aggregate(g, src_p, dst_p, zero_rows):
    n, d = g.shape
    e = dst_p.shape[0]
    w = 160
    assert e % w == 0
    nwin = e // w
    nw_ = NC * NS
    wpass = -(-nwin // nw_)
    nch = n // CH
    npass = -(-nch // NS)

    @functools.partial(
        pl.kernel,
        out_type=jax.ShapeDtypeStruct((NC, n, d), jnp.float32),
        mesh=_MESH,
        scratch_types=[
            pltpu.VMEM((w,), jnp.int32),
            pltpu.VMEM((w,), jnp.int32),
            pltpu.VMEM((w,), jnp.int32),
            pltpu.VMEM((w,), jnp.int32),
            pltpu.VMEM((w, d), jnp.float32),
            pltpu.VMEM((w, d), jnp.float32),
            pltpu.VMEM_SHARED((n, d), jnp.float32),
            pltpu.SemaphoreType.DMA,
            pltpu.SemaphoreType.DMA,
            pltpu.SemaphoreType.DMA,
            pltpu.SemaphoreType.DMA,
            pltpu.SemaphoreType.DMA,
            pltpu.SemaphoreType.DMA,
            pltpu.SemaphoreType.DMA,
            pltpu.SemaphoreType.DMA,
        ])
    def agg_kernel(g_hbm, src_hbm, dst_hbm, zero_hbm, out_hbm,
                   src0, dst0, src1, dst1, rows0, rows1, acc,
                   ss0, sd0, sg0, sa0, ss1, sd1, sg1, sa1):
        core = lax.axis_index("core")
        sub = lax.axis_index("subcore")
        wid = core * NS + sub

        @pl.loop(0, npass)
        def _(c):
            chunk = c * NS + sub

            @pl.when(chunk < nch)
            def _():
                pltpu.sync_copy(zero_hbm, acc.at[pl.ds(chunk * CH, CH)])

        plsc.subcore_barrier()

        def exists(k):
            return k * nw_ + wid < nwin

        def off(k):
            return (k * nw_ + wid) * w

        def issue_idx(k, src_b, dst_b, ssem, dsem):
            @pl.when(exists(k))
            def _():
                pltpu.async_copy(src_hbm.at[pl.ds(off(k), w)], src_b, ssem)
                pltpu.async_copy(dst_hbm.at[pl.ds(off(k), w)], dst_b, dsem)

        def issue_gather(k, src_b, rows_b, ssem, gsem):
            @pl.when(exists(k))
            def _():
                pltpu.make_async_copy(src_hbm.at[pl.ds(off(k), w)], src_b,
                                      ssem).wait()
                pltpu.async_copy(g_hbm.at[src_b], rows_b, gsem)

        def start_scatter(k, src_b, dst_b, rows_b, dsem, gsem, asem):
            @pl.when(exists(k))
            def _():
                pltpu.make_async_copy(g_hbm.at[src_b], rows_b, gsem).wait()
                pltpu.make_async_copy(dst_hbm.at[pl.ds(off(k), w)], dst_b,
                                      dsem).wait()
                pltpu.async_copy(rows_b, acc.at[dst_b], asem, add=True)

        def wait_scatter(k, dst_b, rows_b, asem):
            @pl.when(exists(k))
            def _():
                pltpu.make_async_copy(rows_b, acc.at[dst_b], asem).wait()

        issue_idx(0, src0, dst0, ss0, sd0)
        issue_idx(1, src1, dst1, ss1, sd1)
        issue_gather(0, src0, rows0, ss0, sg0)

        @pl.loop(0, (wpass + 1) // 2)
        def _(t):
            k = 2 * t
            issue_gather(k + 1, src1, rows1, ss1, sg1)
            start_scatter(k, src0, dst0, rows0, sd0, sg0, sa0)
            wait_scatter(k, dst0, rows0, sa0)
            issue_idx(k + 2, src0, dst0, ss0, sd0)
            issue_gather(k + 2, src0, rows0, ss0, sg0)
            start_scatter(k + 1, src1, dst1, rows1, sd1, sg1, sa1)
            wait_scatter(k + 1, dst1, rows1, sa1)
            issue_idx(k + 3, src1, dst1, ss1, sd1)

        plsc.subcore_barrier()

        @pl.loop(0, npass)
        def _(c):
            chunk = c * NS + sub

            @pl.when(chunk < nch)
            def _():
                pltpu.sync_copy(acc.at[pl.ds(chunk * CH, CH)],
                                out_hbm.at[core].at[pl.ds(chunk * CH, CH)])

    return agg_kernel(g, src_p, dst_p, zero_rows)


def _tc_first(x, w1, dg, blk):
    n, d = x.shape
    h = w1.shape[1]
    nb = n // blk

    def body(x_ref, w_ref, d0_ref, d1_ref, g_ref, dinv_ref):
        deg = d0_ref[0] + d1_ref[0] + 1.0
        dinv = lax.rsqrt(jnp.maximum(deg, 1.0))
        hh = jnp.dot(x_ref[...], w_ref[...],
                     preferred_element_type=jnp.float32, precision=lax.Precision.HIGHEST)
        g_ref[...] = hh * dinv
        dinv_ref[...] = jnp.broadcast_to(dinv, (blk, h))

    return pl.pallas_call(
        body,
        grid=(nb,),
        in_specs=[
            pl.BlockSpec((blk, d), lambda i: (i, 0)),
            pl.BlockSpec((d, h), lambda i: (0, 0)),
            pl.BlockSpec((1, blk, 1), lambda i: (0, i, 0)),
            pl.BlockSpec((1, blk, 1), lambda i: (1, i, 0)),
        ],
        out_specs=[
            pl.BlockSpec((blk, h), lambda i: (i, 0)),
            pl.BlockSpec((blk, h), lambda i: (i, 0)),
        ],
        out_shape=[
            jax.ShapeDtypeStruct((n, h), jnp.float32),
            jax.ShapeDtypeStruct((n, h), jnp.float32),
        ])(x, w1, dg, dg)


def _tc_layer(acc, g_prev, dinv, b, wk, blk):
    n, h = g_prev.shape

    def body(a0_ref, a1_ref, gp_ref, dinv_ref, b_ref, w_ref, g_ref):
        s = (a0_ref[0] + a1_ref[0] + gp_ref[...]) * dinv_ref[...] + b_ref[...]
        xk = jnp.maximum(s, 0.0)
        hh = jnp.dot(xk, w_ref[...], preferred_element_type=jnp.float32, precision=lax.Precision.HIGHEST)
        g_ref[...] = hh * dinv_ref[...]

    return pl.pallas_call(
        body,
        grid=(n // blk,),
        in_specs=[
            pl.BlockSpec((1, blk, h), lambda i: (0, i, 0)),
            pl.BlockSpec((1, blk, h), lambda i: (1, i, 0)),
            pl.BlockSpec((blk, h), lambda i: (i, 0)),
            pl.BlockSpec((blk, h), lambda i: (i, 0)),
            pl.BlockSpec((1, h), lambda i: (0, 0)),
            pl.BlockSpec((h, h), lambda i: (0, 0)),
        ],
        out_specs=pl.BlockSpec((blk, h), lambda i: (i, 0)),
        out_shape=jax.ShapeDtypeStruct((n, h), jnp.float32),
    )(acc, acc, g_prev, dinv, b, wk)


def _tc_final(acc, g4, dinv, b4, batch3d, wl, bl, blk):
    n, h = g4.shape
    nb = n // blk

    def body(a0_ref, a1_ref, g_ref, dinv_ref, b_ref, batch_ref,
             wl_ref, bl_ref, out_ref, pooled_ref):
        i = pl.program_id(0)
        h4 = (a0_ref[0] + a1_ref[0] + g_ref[...]) * dinv_ref[...] + b_ref[...]
        ids = jnp.broadcast_to(batch_ref[0], (G, blk))
        onehot = (lax.broadcasted_iota(jnp.int32, (G, blk), 0)
                  == ids).astype(jnp.float32)
        part = jnp.dot(onehot, h4, preferred_element_type=jnp.float32, precision=lax.Precision.HIGHEST)

        @pl.when(i == 0)
        def _():
            pooled_ref[...] = part

        @pl.when(i > 0)
        def _():
            pooled_ref[...] += part

        @pl.when(i == nb - 1)
        def _():
            out_ref[...] = (jnp.dot(pooled_ref[...], wl_ref[...],
                                    preferred_element_type=jnp.float32, precision=lax.Precision.HIGHEST)
                            + bl_ref[...])

    return pl.pallas_call(
        body,
        grid=(nb,),
        in_specs=[
            pl.BlockSpec((1, blk, h), lambda i: (0, i, 0)),
            pl.BlockSpec((1, blk, h), lambda i: (1, i, 0)),
            pl.BlockSpec((blk, h), lambda i: (i, 0)),
            pl.BlockSpec((blk, h), lambda i: (i, 0)),
            pl.BlockSpec((1, h), lambda i: (0, 0)),
            pl.BlockSpec((1, 1, blk), lambda i: (i, 0, 0)),
            pl.BlockSpec((h, 1), lambda i: (0, 0)),
            pl.BlockSpec((1, 1), lambda i: (0, 0)),
        ],
        out_specs=pl.BlockSpec((G, 1), lambda i: (0, 0)),
        out_shape=jax.ShapeDtypeStruct((G, 1), jnp.float32),
        scratch_shapes=[pltpu.VMEM((G, h), jnp.float32)],
    )(acc, acc, g4, dinv, b4, batch3d, wl, bl)


def kernel(x, edge_index, batch, W1, b1, W2, b2, W3, b3, W4, b4, Wl, bl):
    n, d = x.shape
    h = W1.shape[1]
    e = edge_index.shape[1]
    src = edge_index[0]
    dst = edge_index[1]

    blk = 1000 if n % 1000 == 0 else 8
    n_pad = NS * 16 * (-(-n // (NS * 16)))

    zero_rows = jnp.zeros((CH, d), jnp.float32)
    batch3d = batch.reshape(n // blk, 1, blk)
    b1r = b1.reshape(1, h)
    b2r = b2.reshape(1, h)
    b3r = b3.reshape(1, h)
    b4r = b4.reshape(1, h)
    blr = bl.reshape(1, 1)

    dg = _sc_degree(dst, n_pad)
    dg3 = dg.reshape(NC, n_pad, 1)[:, :n]
    g1, dinv = _tc_first(x, W1, dg3, blk)
    a1_ = _sc_aggregate(g1, src, dst, zero_rows)
    g2 = _tc_layer(a1_, g1, dinv, b1r, W2, blk)
    a2_ = _sc_aggregate(g2, src, dst, zero_rows)
    g3 = _tc_layer(a2_, g2, dinv, b2r, W3, blk)
    a3_ = _sc_aggregate(g3, src, dst, zero_rows)
    g4 = _tc_layer(a3_, g3, dinv, b3r, W4, blk)
    a4_ = _sc_aggregate(g4, src, dst, zero_rows)
    return _tc_final(a4_, g4, dinv, b4r, batch3d, Wl, blr, blk)


# speedup vs baseline: 1.2500x; 1.0227x over previous
import dataclasses
import functools

import jax
import jax.numpy as jnp
from jax import lax
from jax.experimental import pallas as pl
from jax.experimental.pallas import tpu as pltpu
from jax.experimental.pallas import tpu_sc as plsc

NC = 2
NS = 16
G = 64
DW = 16
CH = 400

_MESH = plsc.VectorSubcoreMesh(
    core_axis_name="core", subcore_axis_name="subcore",
    num_cores=NC, num_subcores=NS)


def _edge_window(e):
    for w in (256, 128, 64, 32, 16, 8):
        if e % w == 0:
            return w
    return 8


def _sc_degree(dst, n_pad):
    e = dst.shape[0]
    epw = e // (NC * NS)
    rng = n_pad // NS
    nred = rng // 16

    cp = pltpu.CompilerParams()
    if "needs_layout_passes" in pltpu.CompilerParams.__dataclass_fields__:
        cp = dataclasses.replace(cp, needs_layout_passes=False)

    @functools.partial(
        pl.kernel,
        out_type=jax.ShapeDtypeStruct((NC, n_pad), jnp.float32),
        mesh=_MESH,
        compiler_params=cp,
        scratch_types=[
            pltpu.VMEM((epw,), jnp.int32),
            pltpu.VMEM((8, n_pad), jnp.float32),
            pltpu.VMEM((rng,), jnp.float32),
            pltpu.VMEM((rng,), jnp.float32),
            pltpu.VMEM_SHARED((NS, n_pad), jnp.float32),
            pltpu.SemaphoreType.DMA,
        ])
    def deg_kernel(dst_hbm, out_hbm,
                   didx, hist8, acc_r, tmp_r, stage, sem):
        core = lax.axis_index("core")
        sub = lax.axis_index("subcore")
        wid = core * NS + sub
        ones16 = jnp.ones((16,), jnp.float32)
        zeros16 = jnp.zeros((16,), jnp.float32)
        lane = lax.iota(jnp.int32, 16)
        row = lane % 8
        lo = lane < 8
        hi = jnp.logical_not(lo)

        @pl.loop(0, 8)
        def _(b):
            @pl.loop(0, n_pad // 16)
            def _(c):
                hist8[b, pl.ds(c * 16, 16)] = zeros16

        pltpu.sync_copy(dst_hbm.at[pl.ds(wid * epw, epw)], didx)

        @pl.loop(0, epw // 16)
        def _(i):
            idxv = didx[pl.ds(i * 16, 16)]
            plsc.addupdate_scatter(hist8, [row, idxv], ones16, mask=lo)
            plsc.addupdate_scatter(hist8, [row, idxv], ones16, mask=hi)

        @pl.loop(1, 8)
        def _(b):
            @pl.loop(0, n_pad // 16)
            def _(c):
                s = pl.ds(c * 16, 16)
                hist8[0, s] = hist8[0, s] + hist8[b, s]

        pltpu.sync_copy(hist8.at[0], stage.at[sub])
        plsc.subcore_barrier()

        pltpu.sync_copy(stage.at[0].at[pl.ds(sub * rng, rng)], acc_r)

        @pl.loop(1, NS)
        def _(j):
            pltpu.sync_copy(stage.at[j].at[pl.ds(sub * rng, rng)], tmp_r)

            @pl.loop(0, nred)
            def _(c):
                s = pl.ds(c * 16, 16)
                acc_r[s] = acc_r[s] + tmp_r[s]

        pltpu.sync_copy(acc_r, out_hbm.at[core].at[pl.ds(sub * rng, rng)])

    return deg_kernel(dst)


def _sc_aggregate(g, src_p, dst_p, zero_rows):
    n, d = g.shape
    e = dst_p.shape[0]
    w = 160
    assert e % w == 0
    nwin = e // w
    nw_ = NC * NS
    wpass = -(-nwin // nw_)
    nch = n // CH
    npass = -(-nch // NS)

    @functools.partial(
        pl.kernel,
        out_type=jax.ShapeDtypeStruct((NC, n, d), jnp.float32),
        mesh=_MESH,
        scratch_types=[
            pltpu.VMEM((w,), jnp.int32),
            pltpu.VMEM((w,), jnp.int32),
            pltpu.VMEM((w,), jnp.int32),
            pltpu.VMEM((w,), jnp.int32),
            pltpu.VMEM((w, d), jnp.float32),
            pltpu.VMEM((w, d), jnp.float32),
            pltpu.VMEM_SHARED((n, d), jnp.float32),
            pltpu.SemaphoreType.DMA,
            pltpu.SemaphoreType.DMA,
            pltpu.SemaphoreType.DMA,
            pltpu.SemaphoreType.DMA,
            pltpu.SemaphoreType.DMA,
            pltpu.SemaphoreType.DMA,
            pltpu.SemaphoreType.DMA,
            pltpu.SemaphoreType.DMA,
        ])
    def agg_kernel(g_hbm, src_hbm, dst_hbm, zero_hbm, out_hbm,
                   src0, dst0, src1, dst1, rows0, rows1, acc,
                   ss0, sd0, sg0, sa0, ss1, sd1, sg1, sa1):
        core = lax.axis_index("core")
        sub = lax.axis_index("subcore")
        wid = core * NS + sub

        @pl.loop(0, npass)
        def _(c):
            chunk = c * NS + sub

            @pl.when(chunk < nch)
            def _():
                pltpu.sync_copy(zero_hbm, acc.at[pl.ds(chunk * CH, CH)])

        plsc.subcore_barrier()

        def exists(k):
            return k * nw_ + wid < nwin

        def off(k):
            return (k * nw_ + wid) * w

        def issue_idx(k, src_b, dst_b, ssem, dsem):
            @pl.when(exists(k))
            def _():
                pltpu.async_copy(src_hbm.at[pl.ds(off(k), w)], src_b, ssem)
                pltpu.async_copy(dst_hbm.at[pl.ds(off(k), w)], dst_b, dsem)

        def issue_gather(k, src_b, rows_b, ssem, gsem):
            @pl.when(exists(k))
            def _():
                pltpu.make_async_copy(src_hbm.at[pl.ds(off(k), w)], src_b,
                                      ssem).wait()
                pltpu.async_copy(g_hbm.at[src_b], rows_b, gsem)

        def start_scatter(k, src_b, dst_b, rows_b, dsem, gsem, asem):
            @pl.when(exists(k))
            def _():
                pltpu.make_async_copy(g_hbm.at[src_b], rows_b, gsem).wait()
                pltpu.make_async_copy(dst_hbm.at[pl.ds(off(k), w)], dst_b,
                                      dsem).wait()
                pltpu.async_copy(rows_b, acc.at[dst_b], asem, add=True)

        def wait_scatter(k, dst_b, rows_b, asem):
            @pl.when(exists(k))
            def _():
                pltpu.make_async_copy(rows_b, acc.at[dst_b], asem).wait()

        issue_idx(0, src0, dst0, ss0, sd0)
        issue_idx(1, src1, dst1, ss1, sd1)
        issue_gather(0, src0, rows0, ss0, sg0)

        @pl.loop(0, (wpass + 1) // 2)
        def _(t):
            k = 2 * t
            issue_gather(k + 1, src1, rows1, ss1, sg1)
            start_scatter(k, src0, dst0, rows0, sd0, sg0, sa0)
            wait_scatter(k, dst0, rows0, sa0)
            issue_idx(k + 2, src0, dst0, ss0, sd0)
            issue_gather(k + 2, src0, rows0, ss0, sg0)
            start_scatter(k + 1, src1, dst1, rows1, sd1, sg1, sa1)
            wait_scatter(k + 1, dst1, rows1, sa1)
            issue_idx(k + 3, src1, dst1, ss1, sd1)

        plsc.subcore_barrier()

        @pl.loop(0, npass)
        def _(c):
            chunk = c * NS + sub

            @pl.when(chunk < nch)
            def _():
                pltpu.sync_copy(acc.at[pl.ds(chunk * CH, CH)],
                                out_hbm.at[core].at[pl.ds(chunk * CH, CH)])

    return agg_kernel(g, src_p, dst_p, zero_rows)


def _tc_first(x, w1, dg, blk):
    n, d = x.shape
    h = w1.shape[1]
    nb = n // blk

    def body(x_ref, w_ref, d0_ref, d1_ref, g_ref, dinv_ref):
        deg = d0_ref[0] + d1_ref[0] + 1.0
        dinv = lax.rsqrt(jnp.maximum(deg, 1.0))
        hh = jnp.dot(x_ref[...], w_ref[...],
                     preferred_element_type=jnp.float32)
        g_ref[...] = hh * dinv
        dinv_ref[...] = jnp.broadcast_to(dinv, (blk, h))

    return pl.pallas_call(
        body,
        grid=(nb,),
        in_specs=[
            pl.BlockSpec((blk, d), lambda i: (i, 0)),
            pl.BlockSpec((d, h), lambda i: (0, 0)),
            pl.BlockSpec((1, blk, 1), lambda i: (0, i, 0)),
            pl.BlockSpec((1, blk, 1), lambda i: (1, i, 0)),
        ],
        out_specs=[
            pl.BlockSpec((blk, h), lambda i: (i, 0)),
            pl.BlockSpec((blk, h), lambda i: (i, 0)),
        ],
        out_shape=[
            jax.ShapeDtypeStruct((n, h), jnp.float32),
            jax.ShapeDtypeStruct((n, h), jnp.float32),
        ])(x, w1, dg, dg)


def _tc_layer(acc, g_prev, dinv, b, wk, blk):
    n, h = g_prev.shape

    def body(a0_ref, a1_ref, gp_ref, dinv_ref, b_ref, w_ref, g_ref):
        s = (a0_ref[0] + a1_ref[0] + gp_ref[...]) * dinv_ref[...] + b_ref[...]
        xk = jnp.maximum(s, 0.0)
        hh = jnp.dot(xk, w_ref[...], preferred_element_type=jnp.float32)
        g_ref[...] = hh * dinv_ref[...]

    return pl.pallas_call(
        body,
        grid=(n // blk,),
        in_specs=[
            pl.BlockSpec((1, blk, h), lambda i: (0, i, 0)),
            pl.BlockSpec((1, blk, h), lambda i: (1, i, 0)),
            pl.BlockSpec((blk, h), lambda i: (i, 0)),
            pl.BlockSpec((blk, h), lambda i: (i, 0)),
            pl.BlockSpec((1, h), lambda i: (0, 0)),
            pl.BlockSpec((h, h), lambda i: (0, 0)),
        ],
        out_specs=pl.BlockSpec((blk, h), lambda i: (i, 0)),
        out_shape=jax.ShapeDtypeStruct((n, h), jnp.float32),
    )(acc, acc, g_prev, dinv, b, wk)


def _tc_final(acc, g4, dinv, b4, batch3d, wl, bl, blk):
    n, h = g4.shape
    nb = n // blk

    def body(a0_ref, a1_ref, g_ref, dinv_ref, b_ref, batch_ref,
             wl_ref, bl_ref, out_ref, pooled_ref):
        i = pl.program_id(0)
        h4 = (a0_ref[0] + a1_ref[0] + g_ref[...]) * dinv_ref[...] + b_ref[...]
        ids = jnp.broadcast_to(batch_ref[0], (G, blk))
        onehot = (lax.broadcasted_iota(jnp.int32, (G, blk), 0)
                  == ids).astype(jnp.float32)
        part = jnp.dot(onehot, h4, preferred_element_type=jnp.float32, precision=lax.Precision.HIGHEST)

        @pl.when(i == 0)
        def _():
            pooled_ref[...] = part

        @pl.when(i > 0)
        def _():
            pooled_ref[...] += part

        @pl.when(i == nb - 1)
        def _():
            out_ref[...] = (jnp.dot(pooled_ref[...], wl_ref[...],
                                    preferred_element_type=jnp.float32)
                            + bl_ref[...])

    return pl.pallas_call(
        body,
        grid=(nb,),
        in_specs=[
            pl.BlockSpec((1, blk, h), lambda i: (0, i, 0)),
            pl.BlockSpec((1, blk, h), lambda i: (1, i, 0)),
            pl.BlockSpec((blk, h), lambda i: (i, 0)),
            pl.BlockSpec((blk, h), lambda i: (i, 0)),
            pl.BlockSpec((1, h), lambda i: (0, 0)),
            pl.BlockSpec((1, 1, blk), lambda i: (i, 0, 0)),
            pl.BlockSpec((h, 1), lambda i: (0, 0)),
            pl.BlockSpec((1, 1), lambda i: (0, 0)),
        ],
        out_specs=pl.BlockSpec((G, 1), lambda i: (0, 0)),
        out_shape=jax.ShapeDtypeStruct((G, 1), jnp.float32),
        scratch_shapes=[pltpu.VMEM((G, h), jnp.float32)],
    )(acc, acc, g4, dinv, b4, batch3d, wl, bl)


def kernel(x, edge_index, batch, W1, b1, W2, b2, W3, b3, W4, b4, Wl, bl):
    n, d = x.shape
    h = W1.shape[1]
    e = edge_index.shape[1]
    src = edge_index[0]
    dst = edge_index[1]

    blk = 1000 if n % 1000 == 0 else 8
    n_pad = NS * 16 * (-(-n // (NS * 16)))

    zero_rows = jnp.zeros((CH, d), jnp.float32)
    batch3d = batch.reshape(n // blk, 1, blk)
    b1r = b1.reshape(1, h)
    b2r = b2.reshape(1, h)
    b3r = b3.reshape(1, h)
    b4r = b4.reshape(1, h)
    blr = bl.reshape(1, 1)

    dg = _sc_degree(dst, n_pad)
    dg3 = dg.reshape(NC, n_pad, 1)[:, :n]
    g1, dinv = _tc_first(x, W1, dg3, blk)
    a1_ = _sc_aggregate(g1, src, dst, zero_rows)
    g2 = _tc_layer(a1_, g1, dinv, b1r, W2, blk)
    a2_ = _sc_aggregate(g2, src, dst, zero_rows)
    g3 = _tc_layer(a2_, g2, dinv, b2r, W3, blk)
    a3_ = _sc_aggregate(g3, src, dst, zero_rows)
    g4 = _tc_layer(a3_, g3, dinv, b3r, W4, blk)
    a4_ = _sc_aggregate(g4, src, dst, zero_rows)
    return _tc_final(a4_, g4, dinv, b4r, batch3d, Wl, blr, blk)
